# baseline (speedup 1.0000x reference)
import functools

import jax
import jax.numpy as jnp
from jax.experimental import pallas as pl
from jax.experimental.pallas import tpu as pltpu

_CH = 128


def _fold_bn(w, gamma, beta, mean, var, eps=1e-5):
    scale = gamma / jnp.sqrt(var + eps)
    w_eff = w * scale.reshape((-1,) + (1,) * (w.ndim - 1))
    b_eff = beta - mean * scale
    return w_eff, b_eff


def _fused_kernel(x_ref, w1_ref, b1_ref, w2b_ref, b2b_ref, o_ref,
                  xp_ref, sbl_ref, sbc_ref, sbr_ref, *,
                  cin, c1, H, W, pad, ch):
    HW = H * W
    nck = HW // ch
    rpc = ch // W
    w_idx = jax.lax.broadcasted_iota(jnp.int32, (c1, ch), 1) % W
    mask_l = w_idx > 0
    mask_r = w_idx < W - 1

    y1 = jnp.dot(w1_ref[...], x_ref[...].reshape(cin, HW),
                 preferred_element_type=jnp.float32)
    y1 = jnp.maximum(y1 + b1_ref[...], 0.0)
    o_ref[0:c1] = y1.reshape(c1, H, W).astype(o_ref.dtype)
    xp_ref[:, pad:pad + HW] = y1
    xp_ref[:, pad - W:pad] = jnp.zeros((c1, W), jnp.float32)
    xp_ref[:, pad + HW:pad + HW + W] = jnp.zeros((c1, W), jnp.float32)

    def wb(t):
        return w2b_ref[:, t * ch:(t + 1) * ch]

    for c in range(nck):
        base = pad + c * ch
        up = xp_ref[:, base - W:base - W + ch]
        md = xp_ref[:, base:base + ch]
        dn = xp_ref[:, base + W:base + W + ch]
        sbl_ref[:, base:base + ch] = wb(0) * up + wb(3) * md + wb(6) * dn
        sbc_ref[:, base:base + ch] = wb(1) * up + wb(4) * md + wb(7) * dn
        sbr_ref[:, base:base + ch] = wb(2) * up + wb(5) * md + wb(8) * dn

    for c in range(nck):
        base = pad + c * ch
        bl = sbl_ref[:, base - 1:base - 1 + ch]
        bc = sbc_ref[:, base:base + ch]
        br = sbr_ref[:, base + 1:base + 1 + ch]
        y2 = (bc
              + jnp.where(mask_l, bl, 0.0)
              + jnp.where(mask_r, br, 0.0))
        y2 = jnp.maximum(y2 + b2b_ref[:, 0:ch], 0.0)
        o_ref[c1:2 * c1, c * rpc:(c + 1) * rpc, :] = (
            y2.reshape(c1, rpc, W).astype(o_ref.dtype))


def kernel(x, w_primary, bn1_gamma, bn1_beta, bn1_mean, bn1_var,
           w_dw, bn2_gamma, bn2_beta, bn2_mean, bn2_var):
    B, cin, H, W = x.shape
    HW = H * W
    c1 = w_primary.shape[0]
    ch = _CH if HW % _CH == 0 and _CH % W == 0 else HW
    pad = 128

    w1, b1 = _fold_bn(w_primary.reshape(c1, cin),
                      bn1_gamma, bn1_beta, bn1_mean, bn1_var)
    w2, b2 = _fold_bn(w_dw.reshape(c1, 9),
                      bn2_gamma, bn2_beta, bn2_mean, bn2_var)
    w1 = w1.astype(jnp.float32)
    b1 = b1.reshape(c1, 1).astype(jnp.float32)
    w2b = jnp.repeat(w2.astype(jnp.float32), ch, axis=1)
    b2b = jnp.broadcast_to(b2.reshape(c1, 1).astype(jnp.float32), (c1, ch))

    lin = HW + 2 * pad
    out = pl.pallas_call(
        functools.partial(_fused_kernel, cin=cin, c1=c1, H=H, W=W,
                          pad=pad, ch=ch),
        out_shape=jax.ShapeDtypeStruct((B, 2 * c1, H, W), x.dtype),
        grid=(B,),
        in_specs=[
            pl.BlockSpec((None, cin, H, W), lambda b: (b, 0, 0, 0)),
            pl.BlockSpec((c1, cin), lambda b: (0, 0)),
            pl.BlockSpec((c1, 1), lambda b: (0, 0)),
            pl.BlockSpec((c1, 9 * ch), lambda b: (0, 0)),
            pl.BlockSpec((c1, ch), lambda b: (0, 0)),
        ],
        out_specs=pl.BlockSpec((None, 2 * c1, H, W), lambda b: (b, 0, 0, 0)),
        scratch_shapes=[pltpu.VMEM((c1, lin), jnp.float32),
                        pltpu.VMEM((c1, lin), jnp.float32),
                        pltpu.VMEM((c1, lin), jnp.float32),
                        pltpu.VMEM((c1, lin), jnp.float32)],
        compiler_params=pltpu.CompilerParams(
            dimension_semantics=("parallel",)),
        cost_estimate=pl.CostEstimate(
            flops=int(2 * B * HW * cin * c1 + 2 * B * c1 * HW * 9),
            transcendentals=0,
            bytes_accessed=int(4 * (B * cin * HW + B * 2 * c1 * HW))),
    )(x, w1, b1, w2b, b2b)
    return out


# speedup vs baseline: 7.0660x; 7.0660x over previous
import functools

import jax
import jax.numpy as jnp
from jax.experimental import pallas as pl
from jax.experimental.pallas import tpu as pltpu

_G = 1
_CHS = 128


def _fold_bn(w, gamma, beta, mean, var, eps=1e-5):
    scale = gamma / jnp.sqrt(var + eps)
    w_eff = w * scale.reshape((-1,) + (1,) * (w.ndim - 1))
    b_eff = beta - mean * scale
    return w_eff, b_eff


def _fused_kernel(x_ref, w1t_ref, b1b_ref, w2b_ref, b2b_ref, o_ref,
                  xp_ref, sbl_ref, sbc_ref, sbr_ref, *,
                  g, c1, HW, W, chs):
    nck = HW // chs
    sp = HW + 2 * W
    pad2 = 8
    s_idx = jax.lax.broadcasted_iota(jnp.int32, (chs, c1), 0) % W
    mask_l = s_idx > 0
    mask_r = s_idx < W - 1
    w2v = w2b_ref[...]

    y1 = jnp.dot(x_ref[...], w1t_ref[...],
                 preferred_element_type=jnp.float32)
    y1 = jnp.maximum(y1 + b1b_ref[0:1, :], 0.0)
    o_ref[:, 0:c1] = y1.astype(o_ref.dtype)
    for i in range(g):
        base = i * sp + W
        xp_ref[base - W:base, :] = jnp.zeros((W, c1), jnp.float32)
        xp_ref[base:base + HW, :] = y1[i * HW:(i + 1) * HW, :]
        xp_ref[base + HW:base + HW + W, :] = jnp.zeros((W, c1), jnp.float32)

    def wt(t):
        return w2v[0:1, t * c1:(t + 1) * c1]

    for i in range(g):
        for c in range(nck):
            base = i * sp + W + c * chs
            up = xp_ref[base - W:base - W + chs, :]
            md = xp_ref[base:base + chs, :]
            dn = xp_ref[base + W:base + W + chs, :]
            sb = pad2 + i * HW + c * chs
            sbl_ref[sb:sb + chs, :] = wt(0) * up + wt(3) * md + wt(6) * dn
            sbc_ref[sb:sb + chs, :] = wt(1) * up + wt(4) * md + wt(7) * dn
            sbr_ref[sb:sb + chs, :] = wt(2) * up + wt(5) * md + wt(8) * dn

    for i in range(g):
        for c in range(nck):
            sb = pad2 + i * HW + c * chs
            bl = sbl_ref[sb - 1:sb - 1 + chs, :]
            bc = sbc_ref[sb:sb + chs, :]
            br = sbr_ref[sb + 1:sb + 1 + chs, :]
            y2 = (bc
                  + jnp.where(mask_l, bl, 0.0)
                  + jnp.where(mask_r, br, 0.0))
            y2 = jnp.maximum(y2 + b2b_ref[0:1, :], 0.0)
            rb = i * HW + c * chs
            o_ref[rb:rb + chs, c1:2 * c1] = y2.astype(o_ref.dtype)


def kernel(x, w_primary, bn1_gamma, bn1_beta, bn1_mean, bn1_var,
           w_dw, bn2_gamma, bn2_beta, bn2_mean, bn2_var):
    B, cin, H, W = x.shape
    HW = H * W
    c1 = w_primary.shape[0]
    G = _G
    while B % G:
        G //= 2
    chs = _CHS if HW % _CHS == 0 and _CHS % W == 0 else HW

    w1, b1 = _fold_bn(w_primary.reshape(c1, cin),
                      bn1_gamma, bn1_beta, bn1_mean, bn1_var)
    w2, b2 = _fold_bn(w_dw.reshape(c1, 9),
                      bn2_gamma, bn2_beta, bn2_mean, bn2_var)
    w1t = w1.T.astype(jnp.float32)
    b1b = jnp.broadcast_to(b1.astype(jnp.float32), (8, c1))
    w2b = jnp.broadcast_to(w2.T.astype(jnp.float32).reshape(1, 9 * c1),
                           (8, 9 * c1))
    b2b = jnp.broadcast_to(b2.astype(jnp.float32), (8, c1))

    xt = jnp.transpose(x, (0, 2, 3, 1)).reshape(B // G, G * HW, cin)
    out = pl.pallas_call(
        functools.partial(_fused_kernel, g=G, c1=c1, HW=HW, W=W, chs=chs),
        out_shape=jax.ShapeDtypeStruct((B // G, G * HW, 2 * c1), x.dtype),
        grid=(B // G,),
        in_specs=[
            pl.BlockSpec((None, G * HW, cin), lambda b: (b, 0, 0)),
            pl.BlockSpec((cin, c1), lambda b: (0, 0)),
            pl.BlockSpec((8, c1), lambda b: (0, 0)),
            pl.BlockSpec((8, 9 * c1), lambda b: (0, 0)),
            pl.BlockSpec((8, c1), lambda b: (0, 0)),
        ],
        out_specs=pl.BlockSpec((None, G * HW, 2 * c1), lambda b: (b, 0, 0)),
        scratch_shapes=[pltpu.VMEM((G * (HW + 2 * W), c1), jnp.float32),
                        pltpu.VMEM((G * HW + 16, c1), jnp.float32),
                        pltpu.VMEM((G * HW + 16, c1), jnp.float32),
                        pltpu.VMEM((G * HW + 16, c1), jnp.float32)],
        compiler_params=pltpu.CompilerParams(
            dimension_semantics=("parallel",)),
        cost_estimate=pl.CostEstimate(
            flops=int(2 * B * HW * cin * c1 + 2 * B * c1 * HW * 9),
            transcendentals=0,
            bytes_accessed=int(4 * (B * cin * HW + B * 2 * c1 * HW))),
    )(xt, w1t, b1b, w2b, b2b)
    return jnp.transpose(out.reshape(B, H, W, 2 * c1), (0, 3, 1, 2))


# speedup vs baseline: 10.7562x; 1.5222x over previous
import functools

import jax
import jax.numpy as jnp
from jax.experimental import pallas as pl
from jax.experimental.pallas import tpu as pltpu

_G = 4
_CHS = 128


def _fold_bn(w, gamma, beta, mean, var, eps=1e-5):
    scale = gamma / jnp.sqrt(var + eps)
    w_eff = w * scale.reshape((-1,) + (1,) * (w.ndim - 1))
    b_eff = beta - mean * scale
    return w_eff, b_eff


def _fused_kernel(x_ref, w1t_ref, b1b_ref, w2b_ref, b2b_ref, o_ref,
                  xp_ref, sbl_ref, sbc_ref, sbr_ref, *,
                  g, c1, HW, W, chs):
    nck = HW // chs
    sp = HW + 2 * W
    pad2 = 8
    s_idx = jax.lax.broadcasted_iota(jnp.int32, (chs, c1), 0) % W
    mask_l = s_idx > 0
    mask_r = s_idx < W - 1
    w2v = w2b_ref[...]

    y1 = jnp.dot(x_ref[...], w1t_ref[...],
                 preferred_element_type=jnp.float32)
    y1 = jnp.maximum(y1 + b1b_ref[0:1, :], 0.0)
    o_ref[:, 0:c1] = y1.astype(o_ref.dtype)
    for i in range(g):
        base = i * sp + W
        xp_ref[base - W:base, :] = jnp.zeros((W, c1), jnp.float32)
        xp_ref[base:base + HW, :] = y1[i * HW:(i + 1) * HW, :]
        xp_ref[base + HW:base + HW + W, :] = jnp.zeros((W, c1), jnp.float32)

    def wt(t):
        return w2v[0:1, t * c1:(t + 1) * c1]

    for i in range(g):
        for c in range(nck):
            base = i * sp + W + c * chs
            up = xp_ref[base - W:base - W + chs, :]
            md = xp_ref[base:base + chs, :]
            dn = xp_ref[base + W:base + W + chs, :]
            sb = pad2 + i * HW + c * chs
            sbl_ref[sb:sb + chs, :] = wt(0) * up + wt(3) * md + wt(6) * dn
            sbc_ref[sb:sb + chs, :] = wt(1) * up + wt(4) * md + wt(7) * dn
            sbr_ref[sb:sb + chs, :] = wt(2) * up + wt(5) * md + wt(8) * dn

    for i in range(g):
        for c in range(nck):
            sb = pad2 + i * HW + c * chs
            bl = sbl_ref[sb - 1:sb - 1 + chs, :]
            bc = sbc_ref[sb:sb + chs, :]
            br = sbr_ref[sb + 1:sb + 1 + chs, :]
            y2 = (bc
                  + jnp.where(mask_l, bl, 0.0)
                  + jnp.where(mask_r, br, 0.0))
            y2 = jnp.maximum(y2 + b2b_ref[0:1, :], 0.0)
            rb = i * HW + c * chs
            o_ref[rb:rb + chs, c1:2 * c1] = y2.astype(o_ref.dtype)


def kernel(x, w_primary, bn1_gamma, bn1_beta, bn1_mean, bn1_var,
           w_dw, bn2_gamma, bn2_beta, bn2_mean, bn2_var):
    B, cin, H, W = x.shape
    HW = H * W
    c1 = w_primary.shape[0]
    G = _G
    while B % G:
        G //= 2
    chs = _CHS if HW % _CHS == 0 and _CHS % W == 0 else HW

    w1, b1 = _fold_bn(w_primary.reshape(c1, cin),
                      bn1_gamma, bn1_beta, bn1_mean, bn1_var)
    w2, b2 = _fold_bn(w_dw.reshape(c1, 9),
                      bn2_gamma, bn2_beta, bn2_mean, bn2_var)
    w1t = w1.T.astype(jnp.float32)
    b1b = jnp.broadcast_to(b1.astype(jnp.float32), (8, c1))
    w2b = jnp.broadcast_to(w2.T.astype(jnp.float32).reshape(1, 9 * c1),
                           (8, 9 * c1))
    b2b = jnp.broadcast_to(b2.astype(jnp.float32), (8, c1))

    xt = jnp.transpose(x, (0, 2, 3, 1)).reshape(B // G, G * HW, cin)
    out = pl.pallas_call(
        functools.partial(_fused_kernel, g=G, c1=c1, HW=HW, W=W, chs=chs),
        out_shape=jax.ShapeDtypeStruct((B // G, G * HW, 2 * c1), x.dtype),
        grid=(B // G,),
        in_specs=[
            pl.BlockSpec((None, G * HW, cin), lambda b: (b, 0, 0)),
            pl.BlockSpec((cin, c1), lambda b: (0, 0)),
            pl.BlockSpec((8, c1), lambda b: (0, 0)),
            pl.BlockSpec((8, 9 * c1), lambda b: (0, 0)),
            pl.BlockSpec((8, c1), lambda b: (0, 0)),
        ],
        out_specs=pl.BlockSpec((None, G * HW, 2 * c1), lambda b: (b, 0, 0)),
        scratch_shapes=[pltpu.VMEM((G * (HW + 2 * W), c1), jnp.float32),
                        pltpu.VMEM((G * HW + 16, c1), jnp.float32),
                        pltpu.VMEM((G * HW + 16, c1), jnp.float32),
                        pltpu.VMEM((G * HW + 16, c1), jnp.float32)],
        compiler_params=pltpu.CompilerParams(
            dimension_semantics=("parallel",)),
        cost_estimate=pl.CostEstimate(
            flops=int(2 * B * HW * cin * c1 + 2 * B * c1 * HW * 9),
            transcendentals=0,
            bytes_accessed=int(4 * (B * cin * HW + B * 2 * c1 * HW))),
    )(xt, w1t, b1b, w2b, b2b)
    return jnp.transpose(out.reshape(B, H, W, 2 * c1), (0, 3, 1, 2))


# speedup vs baseline: 11.5318x; 1.0721x over previous
import functools

import jax
import jax.numpy as jnp
from jax.experimental import pallas as pl
from jax.experimental.pallas import tpu as pltpu

_G = 8
_CHS = 128


def _fold_bn(w, gamma, beta, mean, var, eps=1e-5):
    scale = gamma / jnp.sqrt(var + eps)
    w_eff = w * scale.reshape((-1,) + (1,) * (w.ndim - 1))
    b_eff = beta - mean * scale
    return w_eff, b_eff


def _fused_kernel(x_ref, w1t_ref, b1b_ref, w2b_ref, b2b_ref, o_ref,
                  xp_ref, sbl_ref, sbc_ref, sbr_ref, *,
                  g, c1, HW, W, chs):
    nck = HW // chs
    sp = HW + 2 * W
    pad2 = 8
    s_idx = jax.lax.broadcasted_iota(jnp.int32, (chs, c1), 0) % W
    mask_l = s_idx > 0
    mask_r = s_idx < W - 1
    w2v = w2b_ref[...]

    y1 = jnp.dot(x_ref[...], w1t_ref[...],
                 preferred_element_type=jnp.float32)
    y1 = jnp.maximum(y1 + b1b_ref[0:1, :], 0.0)
    o_ref[:, 0:c1] = y1.astype(o_ref.dtype)
    for i in range(g):
        base = i * sp + W
        xp_ref[base - W:base, :] = jnp.zeros((W, c1), jnp.float32)
        xp_ref[base:base + HW, :] = y1[i * HW:(i + 1) * HW, :]
        xp_ref[base + HW:base + HW + W, :] = jnp.zeros((W, c1), jnp.float32)

    def wt(t):
        return w2v[0:1, t * c1:(t + 1) * c1]

    for i in range(g):
        for c in range(nck):
            base = i * sp + W + c * chs
            up = xp_ref[base - W:base - W + chs, :]
            md = xp_ref[base:base + chs, :]
            dn = xp_ref[base + W:base + W + chs, :]
            sb = pad2 + i * HW + c * chs
            sbl_ref[sb:sb + chs, :] = wt(0) * up + wt(3) * md + wt(6) * dn
            sbc_ref[sb:sb + chs, :] = wt(1) * up + wt(4) * md + wt(7) * dn
            sbr_ref[sb:sb + chs, :] = wt(2) * up + wt(5) * md + wt(8) * dn

    for i in range(g):
        for c in range(nck):
            sb = pad2 + i * HW + c * chs
            bl = sbl_ref[sb - 1:sb - 1 + chs, :]
            bc = sbc_ref[sb:sb + chs, :]
            br = sbr_ref[sb + 1:sb + 1 + chs, :]
            y2 = (bc
                  + jnp.where(mask_l, bl, 0.0)
                  + jnp.where(mask_r, br, 0.0))
            y2 = jnp.maximum(y2 + b2b_ref[0:1, :], 0.0)
            rb = i * HW + c * chs
            o_ref[rb:rb + chs, c1:2 * c1] = y2.astype(o_ref.dtype)


def kernel(x, w_primary, bn1_gamma, bn1_beta, bn1_mean, bn1_var,
           w_dw, bn2_gamma, bn2_beta, bn2_mean, bn2_var):
    B, cin, H, W = x.shape
    HW = H * W
    c1 = w_primary.shape[0]
    G = _G
    while B % G:
        G //= 2
    chs = _CHS if HW % _CHS == 0 and _CHS % W == 0 else HW

    w1, b1 = _fold_bn(w_primary.reshape(c1, cin),
                      bn1_gamma, bn1_beta, bn1_mean, bn1_var)
    w2, b2 = _fold_bn(w_dw.reshape(c1, 9),
                      bn2_gamma, bn2_beta, bn2_mean, bn2_var)
    w1t = w1.T.astype(jnp.float32)
    b1b = jnp.broadcast_to(b1.astype(jnp.float32), (8, c1))
    w2b = jnp.broadcast_to(w2.T.astype(jnp.float32).reshape(1, 9 * c1),
                           (8, 9 * c1))
    b2b = jnp.broadcast_to(b2.astype(jnp.float32), (8, c1))

    xt = jnp.transpose(x, (0, 2, 3, 1)).reshape(B // G, G * HW, cin)
    out = pl.pallas_call(
        functools.partial(_fused_kernel, g=G, c1=c1, HW=HW, W=W, chs=chs),
        out_shape=jax.ShapeDtypeStruct((B // G, G * HW, 2 * c1), x.dtype),
        grid=(B // G,),
        in_specs=[
            pl.BlockSpec((None, G * HW, cin), lambda b: (b, 0, 0)),
            pl.BlockSpec((cin, c1), lambda b: (0, 0)),
            pl.BlockSpec((8, c1), lambda b: (0, 0)),
            pl.BlockSpec((8, 9 * c1), lambda b: (0, 0)),
            pl.BlockSpec((8, c1), lambda b: (0, 0)),
        ],
        out_specs=pl.BlockSpec((None, G * HW, 2 * c1), lambda b: (b, 0, 0)),
        scratch_shapes=[pltpu.VMEM((G * (HW + 2 * W), c1), jnp.float32),
                        pltpu.VMEM((G * HW + 16, c1), jnp.float32),
                        pltpu.VMEM((G * HW + 16, c1), jnp.float32),
                        pltpu.VMEM((G * HW + 16, c1), jnp.float32)],
        compiler_params=pltpu.CompilerParams(
            dimension_semantics=("parallel",)),
        cost_estimate=pl.CostEstimate(
            flops=int(2 * B * HW * cin * c1 + 2 * B * c1 * HW * 9),
            transcendentals=0,
            bytes_accessed=int(4 * (B * cin * HW + B * 2 * c1 * HW))),
    )(xt, w1t, b1b, w2b, b2b)
    return jnp.transpose(out.reshape(B, H, W, 2 * c1), (0, 3, 1, 2))
